# baseline (speedup 1.0000x reference)
import functools
import jax
import jax.numpy as jnp
import numpy as np
from jax import lax
from jax.experimental import pallas as pl
from jax.experimental.pallas import tpu as pltpu
from jax.experimental.pallas import tpu_sc as plsc

_B, _K, _H, _W = 16, 18, 256, 256
_NC, _NS, _L = 2, 16, 16
_NW = _NC * _NS
_PPW = (_B * _K) // _NW
_CH = _H // 2
_NBUF = 3

_mesh = plsc.VectorSubcoreMesh(core_axis_name="c", subcore_axis_name="s",
                               num_cores=_NC, num_subcores=_NS)


def _lane_tables():
    du, dv, mv = [], [], []
    for v in range(8):
        l = lax.iota(jnp.int32, _L) + (v * _L)
        u = lax.shift_right_logical(l * 745, 13)
        du.append(u)
        dv.append(l - u * 11)
        mv.append(l < 121)
    return du, dv, mv


def _chunk_vectors(tables, xr_v, xc_v, pid, half):
    du, dv, mv = tables
    pidv = jnp.full((_L,), pid, jnp.int32)
    r0 = plsc.load_gather(xr_v, [pidv])
    c0 = plsc.load_gather(xc_v, [pidv])
    base = half * _CH
    vecs = []
    for v in range(8):
        r = jnp.minimum(jnp.maximum(r0 + du[v], 0), _H - 1)
        c = jnp.minimum(jnp.maximum(c0 + dv[v], 0), _W - 1)
        m = mv[v] & (r >= base) & (r < base + _CH)
        vecs.append((r - base, c, m))
    return vecs


def _scatter(buf, vecs, val):
    x = jnp.full((_L,), val, jnp.float32)
    for r, c, m in vecs:
        plsc.store_scatter(buf, [r, c], x, mask=m)


def _sc_body(xr_hbm, xc_hbm, out_hbm, xr_v, xc_v, b0, b1, b2, s0, s1, s2):
    wid = lax.axis_index("s") * _NC + lax.axis_index("c")
    pltpu.sync_copy(xr_hbm, xr_v)
    pltpu.sync_copy(xc_hbm, xc_v)
    bufs = (b0, b1, b2)
    sems = (s0, s1, s2)

    zl = jnp.zeros((_L,), jnp.float32)

    def zrow(r, carry):
        for cc in range(_W // _L):
            b0[r, pl.ds(cc * _L, _L)] = zl
            b1[r, pl.ds(cc * _L, _L)] = zl
            b2[r, pl.ds(cc * _L, _L)] = zl
        return carry

    lax.fori_loop(0, _CH, zrow, 0)

    tables = _lane_tables()
    nch = _PPW * 2
    copies = [None] * nch
    hist = [None] * nch
    for j in range(nch):
        buf = bufs[j % _NBUF]
        sem = sems[j % _NBUF]
        if j >= _NBUF:
            copies[j - _NBUF].wait()
            _scatter(buf, hist[j - _NBUF], 0.0)
        pid = wid * _PPW + j // 2
        half = j % 2
        vecs = _chunk_vectors(tables, xr_v, xc_v, pid, half)
        hist[j] = vecs
        _scatter(buf, vecs, 1.0)
        b = lax.shift_right_logical(pid * 1821, 15)
        k = pid - b * _K
        cp = pltpu.make_async_copy(
            buf, out_hbm.at[b, k, pl.ds(half * _CH, _CH)], sem)
        cp.start()
        copies[j] = cp
    for j in range(nch - _NBUF, nch):
        copies[j].wait()


_sc_kernel = functools.partial(
    pl.kernel,
    out_type=jax.ShapeDtypeStruct((_B, _K, _H, _W), jnp.float32),
    mesh=_mesh,
    compiler_params=pltpu.CompilerParams(needs_layout_passes=False),
    scratch_types=[
        pltpu.VMEM((_B * _K,), jnp.int32),
        pltpu.VMEM((_B * _K,), jnp.int32),
        pltpu.VMEM((_CH, _W), jnp.float32),
        pltpu.VMEM((_CH, _W), jnp.float32),
        pltpu.VMEM((_CH, _W), jnp.float32),
        pltpu.SemaphoreType.DMA,
        pltpu.SemaphoreType.DMA,
        pltpu.SemaphoreType.DMA,
    ],
)(_sc_body)


def kernel(x):
    b, k, _ = x.shape
    xr = (x[:, :, 0] - 5).reshape(b * k)
    xc = (x[:, :, 1] - 5).reshape(b * k)
    y = _sc_kernel(xr, xc)
    return jnp.transpose(y, (0, 2, 3, 1))


# speedup vs baseline: 1.0520x; 1.0520x over previous
import functools
import jax
import jax.numpy as jnp
import numpy as np
from jax import lax
from jax.experimental import pallas as pl
from jax.experimental.pallas import tpu as pltpu
from jax.experimental.pallas import tpu_sc as plsc

_B, _K, _H, _W = 16, 18, 256, 256
_NC, _NS, _L = 2, 16, 16
_NW = _NC * _NS
_PPW = (_B * _K) // _NW
_CH = _H // 2
_NBUF = 3

_mesh = plsc.VectorSubcoreMesh(core_axis_name="c", subcore_axis_name="s",
                               num_cores=_NC, num_subcores=_NS)


def _lane_tables():
    du, dv, mv = [], [], []
    for v in range(8):
        l = lax.iota(jnp.int32, _L) + (v * _L)
        u = lax.shift_right_logical(l * 745, 13)
        du.append(u)
        dv.append(l - u * 11)
        mv.append(l < 121)
    return du, dv, mv


def _chunk_vectors(tables, xr_v, xc_v, pid, half):
    du, dv, mv = tables
    pidv = jnp.full((_L,), pid, jnp.int32)
    r0 = plsc.load_gather(xr_v, [pidv])
    c0 = plsc.load_gather(xc_v, [pidv])
    base = half * _CH
    vecs = []
    for v in range(8):
        r = jnp.minimum(jnp.maximum(r0 + du[v], 0), _H - 1)
        c = jnp.minimum(jnp.maximum(c0 + dv[v], 0), _W - 1)
        m = mv[v] & (r >= base) & (r < base + _CH)
        vecs.append((r - base, c, m))
    return vecs


def _scatter(buf, vecs, val):
    x = jnp.full((_L,), val, jnp.float32)
    for r, c, m in vecs:
        plsc.store_scatter(buf, [r, c], x, mask=m)


def _sc_body(xr_hbm, xc_hbm, out_hbm, xr_v, xc_v, b0, b1, b2, s0, s1, s2):
    wid = lax.axis_index("s") * _NC + lax.axis_index("c")
    pltpu.sync_copy(xr_hbm, xr_v)
    pltpu.sync_copy(xc_hbm, xc_v)
    bufs = (b0, b1, b2)
    sems = (s0, s1, s2)

    zl = jnp.zeros((_L,), jnp.float32)

    def _zero(buf):
        def zrow(r, carry):
            for cc in range(_W // _L):
                buf[r, pl.ds(cc * _L, _L)] = zl
            return carry
        lax.fori_loop(0, _CH, zrow, 0)

    tables = _lane_tables()
    nch = _PPW * 2
    copies = [None] * nch
    hist = [None] * nch
    for j in range(nch):
        buf = bufs[j % _NBUF]
        sem = sems[j % _NBUF]
        if j < _NBUF:
            _zero(buf)
        else:
            copies[j - _NBUF].wait()
            _scatter(buf, hist[j - _NBUF], 0.0)
        pid = wid * _PPW + j // 2
        half = j % 2
        vecs = _chunk_vectors(tables, xr_v, xc_v, pid, half)
        hist[j] = vecs
        _scatter(buf, vecs, 1.0)
        b = lax.shift_right_logical(pid * 1821, 15)
        k = pid - b * _K
        cp = pltpu.make_async_copy(
            buf, out_hbm.at[b, k, pl.ds(half * _CH, _CH)], sem)
        cp.start()
        copies[j] = cp
    for j in range(nch - _NBUF, nch):
        copies[j].wait()


_sc_kernel = functools.partial(
    pl.kernel,
    out_type=jax.ShapeDtypeStruct((_B, _K, _H, _W), jnp.float32),
    mesh=_mesh,
    compiler_params=pltpu.CompilerParams(needs_layout_passes=False),
    scratch_types=[
        pltpu.VMEM((_B * _K,), jnp.int32),
        pltpu.VMEM((_B * _K,), jnp.int32),
        pltpu.VMEM((_CH, _W), jnp.float32),
        pltpu.VMEM((_CH, _W), jnp.float32),
        pltpu.VMEM((_CH, _W), jnp.float32),
        pltpu.SemaphoreType.DMA,
        pltpu.SemaphoreType.DMA,
        pltpu.SemaphoreType.DMA,
    ],
)(_sc_body)


def kernel(x):
    b, k, _ = x.shape
    xr = (x[:, :, 0] - 5).reshape(b * k)
    xc = (x[:, :, 1] - 5).reshape(b * k)
    y = _sc_kernel(xr, xc)
    return jnp.transpose(y, (0, 2, 3, 1))
